# baseline (speedup 1.0000x reference)
import functools

import jax
import jax.numpy as jnp
from jax import lax
from jax.experimental import pallas as pl
from jax.experimental.pallas import tpu as pltpu
from jax.experimental.pallas import tpu_sc as plsc

N = 10000
B = 64
DIMS = [(128, 128), (128, 256), (256, 512), (512, 1024)]
D = 1024
EMB = 128

NC = 2
NS = 16
NW = NC * NS
LW = 128
ACC_PAD = 16
ACC_ROWS = 10240
ZR = 320
RPT_N = ACC_ROWS // NS
RB = 1000


@functools.cache
def _sc_agg(rpt, nchunks):
    mesh = plsc.VectorSubcoreMesh(core_axis_name="c", subcore_axis_name="s")

    @functools.partial(
        pl.kernel,
        out_type=[jax.ShapeDtypeStruct((NC, ACC_ROWS, 128), jnp.float32)]
        * nchunks,
        mesh=mesh,
        scratch_types=[
            pltpu.VMEM((rpt // 2, LW), jnp.int32),
            pltpu.VMEM((rpt // 2, LW), jnp.int32),
            pltpu.VMEM((LW, 128), jnp.float32),
            pltpu.VMEM((LW, 128), jnp.float32),
            pltpu.VMEM_SHARED((ACC_ROWS, 128), jnp.float32),
            pltpu.SemaphoreType.DMA,
            pltpu.SemaphoreType.DMA,
            pltpu.SemaphoreType.DMA,
            pltpu.SemaphoreType.DMA,
        ],
    )
    def agg(*refs):
        x_hbms = refs[:nchunks]
        src_hbm, dst_hbm = refs[nchunks], refs[nchunks + 1]
        out_hbms = refs[nchunks + 2:2 * nchunks + 2]
        (src_v, dst_v, rows_a, rows_b, acc,
         sem_a, sem_b, sem_sa, sem_sb) = refs[2 * nchunks + 2:]
        c = lax.axis_index("c")
        s = lax.axis_index("s")
        wid = s * NC + c
        half = rpt // 2
        npairs = half // 2

        zero16 = jnp.zeros((16,), jnp.float32)

        def zrow(i, carry):
            for l in range(128 // 16):
                rows_a[i, pl.ds(l * 16, 16)] = zero16
            return carry

        def wait_g(x_hbm, buf, sem):
            pltpu.make_async_copy(x_hbm.at[pl.ds(0, LW)], buf, sem).wait()

        def wait_s(buf, sem):
            pltpu.make_async_copy(buf, acc.at[dst_v.at[0]], sem).wait()

        for ci, (x_hbm, out_hbm) in enumerate(zip(x_hbms, out_hbms)):
            if ci == 0:
                lax.fori_loop(0, LW, zrow, 0)
            for k in range(RPT_N // LW):
                pltpu.sync_copy(rows_a, acc.at[pl.ds(s * RPT_N + k * LW, LW)])
            plsc.subcore_barrier()

            for h in range(2):
                base = wid * rpt + h * half
                pltpu.sync_copy(src_hbm.at[pl.ds(base, half)], src_v)
                pltpu.sync_copy(dst_hbm.at[pl.ds(base, half)], dst_v)
                pltpu.async_copy(x_hbm.at[src_v.at[0]], rows_a, sem_a)
                pltpu.async_copy(x_hbm.at[src_v.at[1]], rows_b, sem_b)

                def body(t, carry):
                    ja = t * 2
                    wait_g(x_hbm, rows_a, sem_a)
                    pltpu.async_copy(rows_a, acc.at[dst_v.at[ja]], sem_sa,
                                     add=True)
                    wait_g(x_hbm, rows_b, sem_b)
                    pltpu.async_copy(rows_b, acc.at[dst_v.at[ja + 1]], sem_sb,
                                     add=True)

                    @pl.when(t + 1 < npairs)
                    def _():
                        wait_s(rows_a, sem_sa)
                        pltpu.async_copy(x_hbm.at[src_v.at[ja + 2]], rows_a,
                                         sem_a)
                        wait_s(rows_b, sem_sb)
                        pltpu.async_copy(x_hbm.at[src_v.at[ja + 3]], rows_b,
                                         sem_b)

                    return carry

                lax.fori_loop(0, npairs, body, 0)
                wait_s(rows_a, sem_sa)
                wait_s(rows_b, sem_sb)
            plsc.subcore_barrier()
            pltpu.sync_copy(acc.at[pl.ds(s * RPT_N, RPT_N)],
                            out_hbm.at[c, pl.ds(s * RPT_N, RPT_N)])
            if ci + 1 < nchunks:
                lax.fori_loop(0, LW, zrow, 0)
                plsc.subcore_barrier()

    return agg


@functools.cache
def _sc_cnt(rpt):
    mesh = plsc.VectorSubcoreMesh(core_axis_name="c", subcore_axis_name="s")

    @functools.partial(
        pl.kernel,
        out_type=jax.ShapeDtypeStruct((NC, ACC_ROWS, 128), jnp.float32),
        mesh=mesh,
        scratch_types=[
            pltpu.VMEM((rpt, LW), jnp.int32),
            pltpu.VMEM((LW, 128), jnp.float32),
            pltpu.VMEM_SHARED((ACC_ROWS, 128), jnp.float32),
        ],
    )
    def cnt(dst_hbm, out_hbm, dst_v, ones_v, acc):
        c = lax.axis_index("c")
        s = lax.axis_index("s")
        wid = s * NC + c

        def fill(val):
            v = jnp.full((16,), val, jnp.float32)

            def frow(i, carry):
                for l in range(128 // 16):
                    ones_v[i, pl.ds(l * 16, 16)] = v
                return carry

            lax.fori_loop(0, LW, frow, 0)

        fill(0.0)
        for k in range(RPT_N // LW):
            pltpu.sync_copy(ones_v, acc.at[pl.ds(s * RPT_N + k * LW, LW)])
        fill(1.0)
        pltpu.sync_copy(dst_hbm.at[pl.ds(wid * rpt, rpt)], dst_v)
        plsc.subcore_barrier()

        def body(j, carry):
            pltpu.sync_copy(ones_v, acc.at[dst_v.at[j]], add=True)
            return carry

        lax.fori_loop(0, rpt, body, 0)
        plsc.subcore_barrier()
        pltpu.sync_copy(acc.at[pl.ds(s * RPT_N, RPT_N)],
                        out_hbm.at[c, pl.ds(s * RPT_N, RPT_N)])

    return cnt


def _pad_edges(src, dst):
    e = src.shape[0]
    rpt = -(-e // (NW * LW * 8)) * 8
    epad = NW * rpt * LW
    padi = jnp.arange(epad - e, dtype=jnp.int32)
    srcp = jnp.concatenate([src, padi % 16])
    dstp = jnp.concatenate([dst, N + (padi % ACC_PAD)])
    return srcp.reshape(NW * rpt, LW), dstp.reshape(NW * rpt, LW), rpt


@functools.cache
def _layer_tc(di, do):
    nci, nco = di // 128, do // 128
    nrb = N // RB

    def body(*refs):
        agg_refs = refs[:nci]
        x_refs = refs[nci:2 * nci]
        cnt_ref = refs[2 * nci]
        wl_ref, wr_ref, bl_ref = refs[2 * nci + 1:2 * nci + 4]
        out_refs = refs[2 * nci + 4:]
        cntv = cnt_ref[0, :, 0:1] + cnt_ref[1, :, 0:1]
        recip = 1.0 / jnp.maximum(cntv, 1.0)
        aggf = jnp.concatenate([r[0] + r[1] for r in agg_refs], axis=1) * recip
        xf = jnp.concatenate([r[...] for r in x_refs], axis=1)
        out = (lax.dot_general(aggf, wl_ref[...], (((1,), (1,)), ((), ())),
                               preferred_element_type=jnp.float32)
               + lax.dot_general(xf, wr_ref[...], (((1,), (1,)), ((), ())),
                                 preferred_element_type=jnp.float32)
               + bl_ref[...])
        nrm = jnp.sqrt(jnp.sum(out * out, axis=1, keepdims=True))
        out = jnp.maximum(out / jnp.maximum(nrm, 1e-12), 0.0)
        for co in range(nco):
            out_refs[co][...] = out[:, co * 128:(co + 1) * 128]

    in_specs = (
        [pl.BlockSpec((2, RB, 128), lambda r: (0, r, 0))] * nci
        + [pl.BlockSpec((RB, 128), lambda r: (r, 0))] * nci
        + [pl.BlockSpec((2, RB, 128), lambda r: (0, r, 0)),
           pl.BlockSpec((do, di), lambda r: (0, 0)),
           pl.BlockSpec((do, di), lambda r: (0, 0)),
           pl.BlockSpec((1, do), lambda r: (0, 0))]
    )
    out_specs = [pl.BlockSpec((RB, 128), lambda r: (r, 0))] * nco
    return pl.pallas_call(
        body,
        grid=(nrb,),
        in_specs=in_specs,
        out_specs=out_specs,
        out_shape=[jax.ShapeDtypeStruct((N, 128), jnp.float32)] * nco,
    )


@functools.cache
def _attn1(nci):
    nrb = N // RB

    def body(q_ref, bi_ref, *refs):
        x_refs = refs[:nci]
        eq_ref, m_ref = refs[nci], refs[nci + 1]
        r = pl.program_id(0)
        xb = jnp.concatenate([xr[...] for xr in x_refs], axis=1)
        eqb = lax.dot_general(xb, q_ref[...], (((1,), (1,)), ((), ())),
                              preferred_element_type=jnp.float32)
        eq_ref[...] = eqb
        msk = bi_ref[...] == lax.broadcasted_iota(jnp.int32, (RB, B), 1)
        em = jnp.where(msk, eqb, -jnp.inf)
        pmax = jnp.max(em, axis=0, keepdims=True)

        @pl.when(r == 0)
        def _():
            m_ref[...] = jnp.broadcast_to(pmax, (8, B))

        @pl.when(r > 0)
        def _():
            m_ref[...] = jnp.maximum(m_ref[...], pmax)

    in_specs = (
        [pl.BlockSpec((B, D), lambda r: (0, 0)),
         pl.BlockSpec((RB, 1), lambda r: (r, 0))]
        + [pl.BlockSpec((RB, 128), lambda r: (r, 0))] * nci
    )
    out_specs = [pl.BlockSpec((RB, B), lambda r: (r, 0)),
                 pl.BlockSpec((8, B), lambda r: (0, 0))]
    return pl.pallas_call(
        body,
        grid=(nrb,),
        in_specs=in_specs,
        out_specs=out_specs,
        out_shape=[jax.ShapeDtypeStruct((N, B), jnp.float32),
                   jax.ShapeDtypeStruct((8, B), jnp.float32)],
    )


@functools.cache
def _attn2(nci):
    nrb = N // RB

    def body(bi_ref, m_ref, eq_ref, *refs):
        x_refs = refs[:nci]
        r_ref = refs[nci]
        r = pl.program_id(0)
        m = m_ref[0:1, :]
        mm = jnp.where(m < -3e38, 0.0, m)
        eqb = eq_ref[...]
        msk = bi_ref[...] == lax.broadcasted_iota(jnp.int32, (RB, B), 1)
        w = jnp.where(msk, jnp.exp(eqb - mm), 0.0)
        xb = jnp.concatenate([xr[...] for xr in x_refs]
                             + [jnp.ones((RB, 128), jnp.float32)], axis=1)
        rpart = lax.dot_general(w, xb, (((0,), (0,)), ((), ())),
                                preferred_element_type=jnp.float32)

        @pl.when(r == 0)
        def _():
            r_ref[...] = rpart

        @pl.when(r > 0)
        def _():
            r_ref[...] = r_ref[...] + rpart

        @pl.when(r == nrb - 1)
        def _():
            v = r_ref[...]
            den = jnp.maximum(v[:, nci * 128:nci * 128 + 1], 1e-16)
            r_ref[...] = v / den

    in_specs = (
        [pl.BlockSpec((RB, 1), lambda r: (r, 0)),
         pl.BlockSpec((8, B), lambda r: (0, 0)),
         pl.BlockSpec((RB, B), lambda r: (r, 0))]
        + [pl.BlockSpec((RB, 128), lambda r: (r, 0))] * nci
    )
    out_specs = [pl.BlockSpec((B, (nci + 1) * 128), lambda r: (0, 0))]
    return pl.pallas_call(
        body,
        grid=(nrb,),
        in_specs=in_specs,
        out_specs=out_specs,
        out_shape=[jax.ShapeDtypeStruct((B, (nci + 1) * 128), jnp.float32)],
    )


def _sigmoid(x):
    return 1.0 / (1.0 + jnp.exp(-x))


def _lstm_body(q_ref, r_ref, c_ref, wih_ref, whh_ref, bih_ref, bhh_ref,
               h_out, c_out):
    q = q_ref[...]
    rr = r_ref[...]
    gates = []
    for g in range(4):
        wih = wih_ref[g]
        gate = (lax.dot_general(q, wih[:, :D], (((1,), (1,)), ((), ())),
                                preferred_element_type=jnp.float32)
                + lax.dot_general(rr, wih[:, D:], (((1,), (1,)), ((), ())),
                                  preferred_element_type=jnp.float32)
                + lax.dot_general(q, whh_ref[g], (((1,), (1,)), ((), ())),
                                  preferred_element_type=jnp.float32)
                + bih_ref[g:g + 1, :] + bhh_ref[g:g + 1, :])
        gates.append(gate)
    ig, fg, gg, og = gates
    c_new = _sigmoid(fg) * c_ref[...] + _sigmoid(ig) * jnp.tanh(gg)
    h_out[...] = _sigmoid(og) * jnp.tanh(c_new)
    c_out[...] = c_new


_lstm = pl.pallas_call(
    _lstm_body,
    out_shape=[jax.ShapeDtypeStruct((B, D), jnp.float32),
               jax.ShapeDtypeStruct((B, D), jnp.float32)],
)


def _heads_body(q_ref, r_ref, wmu_ref, bmu_ref, wlv_ref, blv_ref, eps_ref,
                z_ref, zmu_ref, zlv_ref):
    q = q_ref[...]
    rr = r_ref[...]

    def head(w_ref, b_ref):
        return (lax.dot_general(q, w_ref[:, :D], (((1,), (1,)), ((), ())),
                                preferred_element_type=jnp.float32)
                + lax.dot_general(rr, w_ref[:, D:], (((1,), (1,)), ((), ())),
                                  preferred_element_type=jnp.float32)
                + b_ref[...])

    zmu = head(wmu_ref, bmu_ref)
    zlv = head(wlv_ref, blv_ref)
    zmu_ref[...] = zmu
    zlv_ref[...] = zlv
    z_ref[...] = eps_ref[...] * jnp.exp(0.5 * zlv) + zmu


_heads = pl.pallas_call(
    _heads_body,
    out_shape=[jax.ShapeDtypeStruct((B, EMB), jnp.float32)] * 3,
)


def kernel(x, edge_attr, edge_index, batch_index, params):
    f32 = jnp.float32
    src2d, dst2d, rpt = _pad_edges(edge_index[0], edge_index[1])

    cnt2 = _sc_cnt(rpt)(dst2d)
    slabs = [x]
    for i, (di, do) in enumerate(DIMS):
        aggs = _sc_agg(rpt, len(slabs))(*slabs, src2d, dst2d)
        if not isinstance(aggs, (tuple, list)):
            aggs = [aggs]
        slabs = _layer_tc(di, do)(
            *aggs, *slabs, cnt2,
            params[f"Wl{i}"], params[f"Wr{i}"],
            params[f"bl{i}"].reshape(1, do))

    bi2d = batch_index.reshape(N, 1)
    wih4 = params["W_ih"].reshape(4, D, 2 * D)
    whh4 = params["W_hh"].reshape(4, D, D)
    bih4 = params["b_ih"].reshape(4, D)
    bhh4 = params["b_hh"].reshape(4, D)
    nci = len(slabs)

    q = jnp.zeros((B, D), f32)
    r = jnp.zeros((B, D), f32)
    cst = jnp.zeros((B, D), f32)
    for _ in range(4):
        q, cst = _lstm(q, r, cst, wih4, whh4, bih4, bhh4)
        eq, mbuf = _attn1(nci)(q, bi2d, *slabs)
        (rfull,) = _attn2(nci)(bi2d, mbuf, eq, *slabs)
        r = rfull[:, :D]

    eps = jax.random.normal(jax.random.key(42), (B, EMB), f32)
    z, zmu, zlv = _heads(q, r, params["Wmu"], params["bmu"].reshape(1, EMB),
                         params["Wlv"], params["blv"].reshape(1, EMB), eps)
    return (z, zmu, zlv)


# speedup vs baseline: 1.2019x; 1.2019x over previous
import functools

import jax
import jax.numpy as jnp
from jax import lax
from jax.experimental import pallas as pl
from jax.experimental.pallas import tpu as pltpu
from jax.experimental.pallas import tpu_sc as plsc

N = 10000
B = 64
DIMS = [(128, 128), (128, 256), (256, 512), (512, 1024)]
D = 1024
EMB = 128

NC = 2
NS = 16
NW = NC * NS
LW = 128
ACC_PAD = 16
ACC_ROWS = 10240
ZR = 320
RPT_N = ACC_ROWS // NS
RB = 1000


@functools.cache
def _sc_agg(rpt, nchunks):
    mesh = plsc.VectorSubcoreMesh(core_axis_name="c", subcore_axis_name="s")

    @functools.partial(
        pl.kernel,
        out_type=[jax.ShapeDtypeStruct((NC, ACC_ROWS, 128), jnp.float32)]
        * nchunks,
        mesh=mesh,
        scratch_types=[
            pltpu.VMEM((rpt // 2, LW), jnp.int32),
            pltpu.VMEM((rpt // 2, LW), jnp.int32),
            pltpu.VMEM((LW, 128), jnp.float32),
            pltpu.VMEM((LW, 128), jnp.float32),
            pltpu.VMEM_SHARED((ACC_ROWS, 128), jnp.float32),
            pltpu.SemaphoreType.DMA,
            pltpu.SemaphoreType.DMA,
            pltpu.SemaphoreType.DMA,
            pltpu.SemaphoreType.DMA,
        ],
    )
    def agg(*refs):
        x_hbms = refs[:nchunks]
        src_hbm, dst_hbm = refs[nchunks], refs[nchunks + 1]
        out_hbms = refs[nchunks + 2:2 * nchunks + 2]
        (src_v, dst_v, rows_a, rows_b, acc,
         sem_a, sem_b, sem_sa, sem_sb) = refs[2 * nchunks + 2:]
        c = lax.axis_index("c")
        s = lax.axis_index("s")
        wid = s * NC + c
        half = rpt // 2
        npairs = half // 2

        zero16 = jnp.zeros((16,), jnp.float32)

        def zrow(i, carry):
            for l in range(128 // 16):
                rows_a[i, pl.ds(l * 16, 16)] = zero16
            return carry

        def wait_g(x_hbm, buf, sem):
            pltpu.make_async_copy(x_hbm.at[pl.ds(0, LW)], buf, sem).wait()

        def wait_s(buf, sem):
            pltpu.make_async_copy(buf, acc.at[dst_v.at[0]], sem).wait()

        for ci, (x_hbm, out_hbm) in enumerate(zip(x_hbms, out_hbms)):
            if ci == 0:
                lax.fori_loop(0, LW, zrow, 0)
            for k in range(RPT_N // LW):
                pltpu.sync_copy(rows_a, acc.at[pl.ds(s * RPT_N + k * LW, LW)])
            plsc.subcore_barrier()

            for h in range(2):
                base = wid * rpt + h * half
                pltpu.sync_copy(src_hbm.at[pl.ds(base, half)], src_v)
                pltpu.sync_copy(dst_hbm.at[pl.ds(base, half)], dst_v)
                pltpu.async_copy(x_hbm.at[src_v.at[0]], rows_a, sem_a)
                pltpu.async_copy(x_hbm.at[src_v.at[1]], rows_b, sem_b)

                def body(t, carry):
                    ja = t * 2
                    wait_g(x_hbm, rows_a, sem_a)
                    pltpu.sync_copy(rows_a, acc.at[dst_v.at[ja]], add=True)

                    @pl.when(t + 1 < npairs)
                    def _():
                        pltpu.async_copy(x_hbm.at[src_v.at[ja + 2]], rows_a,
                                         sem_a)

                    wait_g(x_hbm, rows_b, sem_b)
                    pltpu.sync_copy(rows_b, acc.at[dst_v.at[ja + 1]], add=True)

                    @pl.when(t + 1 < npairs)
                    def _():
                        pltpu.async_copy(x_hbm.at[src_v.at[ja + 3]], rows_b,
                                         sem_b)

                    return carry

                lax.fori_loop(0, npairs, body, 0)
            plsc.subcore_barrier()
            pltpu.sync_copy(acc.at[pl.ds(s * RPT_N, RPT_N)],
                            out_hbm.at[c, pl.ds(s * RPT_N, RPT_N)])
            if ci + 1 < nchunks:
                lax.fori_loop(0, LW, zrow, 0)
                plsc.subcore_barrier()

    return agg


@functools.cache
def _sc_cnt(rpt):
    mesh = plsc.VectorSubcoreMesh(core_axis_name="c", subcore_axis_name="s")

    @functools.partial(
        pl.kernel,
        out_type=jax.ShapeDtypeStruct((NC, ACC_ROWS, 128), jnp.float32),
        mesh=mesh,
        scratch_types=[
            pltpu.VMEM((rpt, LW), jnp.int32),
            pltpu.VMEM((LW, 128), jnp.float32),
            pltpu.VMEM_SHARED((ACC_ROWS, 128), jnp.float32),
        ],
    )
    def cnt(dst_hbm, out_hbm, dst_v, ones_v, acc):
        c = lax.axis_index("c")
        s = lax.axis_index("s")
        wid = s * NC + c

        def fill(val):
            v = jnp.full((16,), val, jnp.float32)

            def frow(i, carry):
                for l in range(128 // 16):
                    ones_v[i, pl.ds(l * 16, 16)] = v
                return carry

            lax.fori_loop(0, LW, frow, 0)

        fill(0.0)
        for k in range(RPT_N // LW):
            pltpu.sync_copy(ones_v, acc.at[pl.ds(s * RPT_N + k * LW, LW)])
        fill(1.0)
        pltpu.sync_copy(dst_hbm.at[pl.ds(wid * rpt, rpt)], dst_v)
        plsc.subcore_barrier()

        def body(j, carry):
            pltpu.sync_copy(ones_v, acc.at[dst_v.at[j]], add=True)
            return carry

        lax.fori_loop(0, rpt, body, 0)
        plsc.subcore_barrier()
        pltpu.sync_copy(acc.at[pl.ds(s * RPT_N, RPT_N)],
                        out_hbm.at[c, pl.ds(s * RPT_N, RPT_N)])

    return cnt


def _pad_edges(src, dst):
    e = src.shape[0]
    rpt = -(-e // (NW * LW * 8)) * 8
    epad = NW * rpt * LW
    padi = jnp.arange(epad - e, dtype=jnp.int32)
    srcp = jnp.concatenate([src, padi % 16])
    dstp = jnp.concatenate([dst, N + (padi % ACC_PAD)])
    return srcp.reshape(NW * rpt, LW), dstp.reshape(NW * rpt, LW), rpt


@functools.cache
def _layer_tc(di, do):
    nci, nco = di // 128, do // 128
    nrb = N // RB

    def body(*refs):
        agg_refs = refs[:nci]
        x_refs = refs[nci:2 * nci]
        cnt_ref = refs[2 * nci]
        wl_ref, wr_ref, bl_ref = refs[2 * nci + 1:2 * nci + 4]
        out_refs = refs[2 * nci + 4:]
        cntv = cnt_ref[0, :, 0:1] + cnt_ref[1, :, 0:1]
        recip = 1.0 / jnp.maximum(cntv, 1.0)
        aggf = jnp.concatenate([r[0] + r[1] for r in agg_refs], axis=1) * recip
        xf = jnp.concatenate([r[...] for r in x_refs], axis=1)
        out = (lax.dot_general(aggf, wl_ref[...], (((1,), (1,)), ((), ())),
                               preferred_element_type=jnp.float32)
               + lax.dot_general(xf, wr_ref[...], (((1,), (1,)), ((), ())),
                                 preferred_element_type=jnp.float32)
               + bl_ref[...])
        nrm = jnp.sqrt(jnp.sum(out * out, axis=1, keepdims=True))
        out = jnp.maximum(out / jnp.maximum(nrm, 1e-12), 0.0)
        for co in range(nco):
            out_refs[co][...] = out[:, co * 128:(co + 1) * 128]

    in_specs = (
        [pl.BlockSpec((2, RB, 128), lambda r: (0, r, 0))] * nci
        + [pl.BlockSpec((RB, 128), lambda r: (r, 0))] * nci
        + [pl.BlockSpec((2, RB, 128), lambda r: (0, r, 0)),
           pl.BlockSpec((do, di), lambda r: (0, 0)),
           pl.BlockSpec((do, di), lambda r: (0, 0)),
           pl.BlockSpec((1, do), lambda r: (0, 0))]
    )
    out_specs = [pl.BlockSpec((RB, 128), lambda r: (r, 0))] * nco
    return pl.pallas_call(
        body,
        grid=(nrb,),
        in_specs=in_specs,
        out_specs=out_specs,
        out_shape=[jax.ShapeDtypeStruct((N, 128), jnp.float32)] * nco,
    )


@functools.cache
def _attn1(nci):
    nrb = N // RB

    def body(q_ref, bi_ref, *refs):
        x_refs = refs[:nci]
        eq_ref, m_ref = refs[nci], refs[nci + 1]
        r = pl.program_id(0)
        xb = jnp.concatenate([xr[...] for xr in x_refs], axis=1)
        eqb = lax.dot_general(xb, q_ref[...], (((1,), (1,)), ((), ())),
                              preferred_element_type=jnp.float32)
        eq_ref[...] = eqb
        msk = bi_ref[...] == lax.broadcasted_iota(jnp.int32, (RB, B), 1)
        em = jnp.where(msk, eqb, -jnp.inf)
        pmax = jnp.max(em, axis=0, keepdims=True)

        @pl.when(r == 0)
        def _():
            m_ref[...] = jnp.broadcast_to(pmax, (8, B))

        @pl.when(r > 0)
        def _():
            m_ref[...] = jnp.maximum(m_ref[...], pmax)

    in_specs = (
        [pl.BlockSpec((B, D), lambda r: (0, 0)),
         pl.BlockSpec((RB, 1), lambda r: (r, 0))]
        + [pl.BlockSpec((RB, 128), lambda r: (r, 0))] * nci
    )
    out_specs = [pl.BlockSpec((RB, B), lambda r: (r, 0)),
                 pl.BlockSpec((8, B), lambda r: (0, 0))]
    return pl.pallas_call(
        body,
        grid=(nrb,),
        in_specs=in_specs,
        out_specs=out_specs,
        out_shape=[jax.ShapeDtypeStruct((N, B), jnp.float32),
                   jax.ShapeDtypeStruct((8, B), jnp.float32)],
    )


@functools.cache
def _attn2(nci):
    nrb = N // RB

    def body(bi_ref, m_ref, eq_ref, *refs):
        x_refs = refs[:nci]
        r_ref = refs[nci]
        r = pl.program_id(0)
        m = m_ref[0:1, :]
        mm = jnp.where(m < -3e38, 0.0, m)
        eqb = eq_ref[...]
        msk = bi_ref[...] == lax.broadcasted_iota(jnp.int32, (RB, B), 1)
        w = jnp.where(msk, jnp.exp(eqb - mm), 0.0)
        xb = jnp.concatenate([xr[...] for xr in x_refs]
                             + [jnp.ones((RB, 128), jnp.float32)], axis=1)
        rpart = lax.dot_general(w, xb, (((0,), (0,)), ((), ())),
                                preferred_element_type=jnp.float32)

        @pl.when(r == 0)
        def _():
            r_ref[...] = rpart

        @pl.when(r > 0)
        def _():
            r_ref[...] = r_ref[...] + rpart

        @pl.when(r == nrb - 1)
        def _():
            v = r_ref[...]
            den = jnp.maximum(v[:, nci * 128:nci * 128 + 1], 1e-16)
            r_ref[...] = v / den

    in_specs = (
        [pl.BlockSpec((RB, 1), lambda r: (r, 0)),
         pl.BlockSpec((8, B), lambda r: (0, 0)),
         pl.BlockSpec((RB, B), lambda r: (r, 0))]
        + [pl.BlockSpec((RB, 128), lambda r: (r, 0))] * nci
    )
    out_specs = [pl.BlockSpec((B, (nci + 1) * 128), lambda r: (0, 0))]
    return pl.pallas_call(
        body,
        grid=(nrb,),
        in_specs=in_specs,
        out_specs=out_specs,
        out_shape=[jax.ShapeDtypeStruct((B, (nci + 1) * 128), jnp.float32)],
    )


def _sigmoid(x):
    return 1.0 / (1.0 + jnp.exp(-x))


def _lstm_body(q_ref, r_ref, c_ref, wih_ref, whh_ref, bih_ref, bhh_ref,
               h_out, c_out):
    q = q_ref[...]
    rr = r_ref[...]
    gates = []
    for g in range(4):
        wih = wih_ref[g]
        gate = (lax.dot_general(q, wih[:, :D], (((1,), (1,)), ((), ())),
                                preferred_element_type=jnp.float32)
                + lax.dot_general(rr, wih[:, D:], (((1,), (1,)), ((), ())),
                                  preferred_element_type=jnp.float32)
                + lax.dot_general(q, whh_ref[g], (((1,), (1,)), ((), ())),
                                  preferred_element_type=jnp.float32)
                + bih_ref[g:g + 1, :] + bhh_ref[g:g + 1, :])
        gates.append(gate)
    ig, fg, gg, og = gates
    c_new = _sigmoid(fg) * c_ref[...] + _sigmoid(ig) * jnp.tanh(gg)
    h_out[...] = _sigmoid(og) * jnp.tanh(c_new)
    c_out[...] = c_new


_lstm = pl.pallas_call(
    _lstm_body,
    out_shape=[jax.ShapeDtypeStruct((B, D), jnp.float32),
               jax.ShapeDtypeStruct((B, D), jnp.float32)],
)


def _heads_body(q_ref, r_ref, wmu_ref, bmu_ref, wlv_ref, blv_ref, eps_ref,
                z_ref, zmu_ref, zlv_ref):
    q = q_ref[...]
    rr = r_ref[...]

    def head(w_ref, b_ref):
        return (lax.dot_general(q, w_ref[:, :D], (((1,), (1,)), ((), ())),
                                preferred_element_type=jnp.float32)
                + lax.dot_general(rr, w_ref[:, D:], (((1,), (1,)), ((), ())),
                                  preferred_element_type=jnp.float32)
                + b_ref[...])

    zmu = head(wmu_ref, bmu_ref)
    zlv = head(wlv_ref, blv_ref)
    zmu_ref[...] = zmu
    zlv_ref[...] = zlv
    z_ref[...] = eps_ref[...] * jnp.exp(0.5 * zlv) + zmu


_heads = pl.pallas_call(
    _heads_body,
    out_shape=[jax.ShapeDtypeStruct((B, EMB), jnp.float32)] * 3,
)


def kernel(x, edge_attr, edge_index, batch_index, params):
    f32 = jnp.float32
    src2d, dst2d, rpt = _pad_edges(edge_index[0], edge_index[1])

    cnt2 = _sc_cnt(rpt)(dst2d)
    slabs = [x]
    for i, (di, do) in enumerate(DIMS):
        aggs = _sc_agg(rpt, len(slabs))(*slabs, src2d, dst2d)
        if not isinstance(aggs, (tuple, list)):
            aggs = [aggs]
        slabs = _layer_tc(di, do)(
            *aggs, *slabs, cnt2,
            params[f"Wl{i}"], params[f"Wr{i}"],
            params[f"bl{i}"].reshape(1, do))

    bi2d = batch_index.reshape(N, 1)
    wih4 = params["W_ih"].reshape(4, D, 2 * D)
    whh4 = params["W_hh"].reshape(4, D, D)
    bih4 = params["b_ih"].reshape(4, D)
    bhh4 = params["b_hh"].reshape(4, D)
    nci = len(slabs)

    q = jnp.zeros((B, D), f32)
    r = jnp.zeros((B, D), f32)
    cst = jnp.zeros((B, D), f32)
    for _ in range(4):
        q, cst = _lstm(q, r, cst, wih4, whh4, bih4, bhh4)
        eq, mbuf = _attn1(nci)(q, bi2d, *slabs)
        (rfull,) = _attn2(nci)(bi2d, mbuf, eq, *slabs)
        r = rfull[:, :D]

    eps = jax.random.normal(jax.random.key(42), (B, EMB), f32)
    z, zmu, zlv = _heads(q, r, params["Wmu"], params["bmu"].reshape(1, EMB),
                         params["Wlv"], params["blv"].reshape(1, EMB), eps)
    return (z, zmu, zlv)


# speedup vs baseline: 1.2027x; 1.0007x over previous
import functools

import jax
import jax.numpy as jnp
from jax import lax
from jax.experimental import pallas as pl
from jax.experimental.pallas import tpu as pltpu
from jax.experimental.pallas import tpu_sc as plsc

N = 10000
B = 64
DIMS = [(128, 128), (128, 256), (256, 512), (512, 1024)]
D = 1024
EMB = 128

NC = 2
NS = 16
NW = NC * NS
LW = 128
ACC_PAD = 16
ACC_ROWS = 10240
ZR = 320
RPT_N = ACC_ROWS // NS
RB = 1000


@functools.cache
def _sc_agg(rpt, nchunks):
    mesh = plsc.VectorSubcoreMesh(core_axis_name="c", subcore_axis_name="s")

    @functools.partial(
        pl.kernel,
        out_type=[jax.ShapeDtypeStruct((NC, ACC_ROWS, 128), jnp.float32)]
        * nchunks,
        mesh=mesh,
        scratch_types=[
            pltpu.VMEM((rpt // 2, LW), jnp.int32),
            pltpu.VMEM((rpt // 2, LW), jnp.int32),
            pltpu.VMEM((LW, 128), jnp.float32),
            pltpu.VMEM((LW, 128), jnp.float32),
            pltpu.VMEM_SHARED((ACC_ROWS, 128), jnp.float32),
            pltpu.SemaphoreType.DMA,
            pltpu.SemaphoreType.DMA,
            pltpu.SemaphoreType.DMA,
            pltpu.SemaphoreType.DMA,
        ],
    )
    def agg(*refs):
        x_hbms = refs[:nchunks]
        src_hbm, dst_hbm = refs[nchunks], refs[nchunks + 1]
        out_hbms = refs[nchunks + 2:2 * nchunks + 2]
        (src_v, dst_v, rows_a, rows_b, acc,
         sem_a, sem_b, sem_sa, sem_sb) = refs[2 * nchunks + 2:]
        c = lax.axis_index("c")
        s = lax.axis_index("s")
        wid = s * NC + c
        half = rpt // 2
        npairs = half // 2

        zero16 = jnp.zeros((16,), jnp.float32)

        def zrow(i, carry):
            for l in range(128 // 16):
                rows_a[i, pl.ds(l * 16, 16)] = zero16
            return carry

        def wait_g(x_hbm, buf, sem):
            pltpu.make_async_copy(x_hbm.at[pl.ds(0, LW)], buf, sem).wait()

        def wait_s(buf, sem):
            pltpu.make_async_copy(buf, acc.at[dst_v.at[0]], sem).wait()

        for ci, (x_hbm, out_hbm) in enumerate(zip(x_hbms, out_hbms)):
            if ci == 0:
                lax.fori_loop(0, LW, zrow, 0)
            for k in range(RPT_N // LW):
                pltpu.sync_copy(rows_a, acc.at[pl.ds(s * RPT_N + k * LW, LW)])
            plsc.subcore_barrier()

            for h in range(2):
                base = wid * rpt + h * half
                pltpu.sync_copy(src_hbm.at[pl.ds(base, half)], src_v)
                pltpu.sync_copy(dst_hbm.at[pl.ds(base, half)], dst_v)
                pltpu.async_copy(x_hbm.at[src_v.at[0]], rows_a, sem_a)
                pltpu.async_copy(x_hbm.at[src_v.at[1]], rows_b, sem_b)

                def body(t, carry):
                    ja = t * 2
                    wait_g(x_hbm, rows_a, sem_a)
                    pltpu.sync_copy(rows_a, acc.at[dst_v.at[ja]], add=True)

                    @pl.when(t + 1 < npairs)
                    def _():
                        pltpu.async_copy(x_hbm.at[src_v.at[ja + 2]], rows_a,
                                         sem_a)

                    wait_g(x_hbm, rows_b, sem_b)
                    pltpu.sync_copy(rows_b, acc.at[dst_v.at[ja + 1]], add=True)

                    @pl.when(t + 1 < npairs)
                    def _():
                        pltpu.async_copy(x_hbm.at[src_v.at[ja + 3]], rows_b,
                                         sem_b)

                    return carry

                lax.fori_loop(0, npairs, body, 0)
            plsc.subcore_barrier()
            pltpu.sync_copy(acc.at[pl.ds(s * RPT_N, RPT_N)],
                            out_hbm.at[c, pl.ds(s * RPT_N, RPT_N)])
            if ci + 1 < nchunks:
                lax.fori_loop(0, LW, zrow, 0)
                plsc.subcore_barrier()

    return agg


@functools.cache
def _sc_cnt(rpt):
    mesh = plsc.VectorSubcoreMesh(core_axis_name="c", subcore_axis_name="s")

    @functools.partial(
        pl.kernel,
        out_type=jax.ShapeDtypeStruct((NC, ACC_ROWS, 128), jnp.float32),
        mesh=mesh,
        scratch_types=[
            pltpu.VMEM((rpt, LW), jnp.int32),
            pltpu.VMEM((LW, 128), jnp.float32),
            pltpu.VMEM_SHARED((ACC_ROWS, 128), jnp.float32),
        ],
    )
    def cnt(dst_hbm, out_hbm, dst_v, ones_v, acc):
        c = lax.axis_index("c")
        s = lax.axis_index("s")
        wid = s * NC + c

        def fill(val):
            v = jnp.full((16,), val, jnp.float32)

            def frow(i, carry):
                for l in range(128 // 16):
                    ones_v[i, pl.ds(l * 16, 16)] = v
                return carry

            lax.fori_loop(0, LW, frow, 0)

        fill(0.0)
        for k in range(RPT_N // LW):
            pltpu.sync_copy(ones_v, acc.at[pl.ds(s * RPT_N + k * LW, LW)])
        fill(1.0)
        pltpu.sync_copy(dst_hbm.at[pl.ds(wid * rpt, rpt)], dst_v)
        plsc.subcore_barrier()

        def body(j, carry):
            pltpu.sync_copy(ones_v, acc.at[dst_v.at[j]], add=True)
            return carry

        lax.fori_loop(0, rpt, body, 0)
        plsc.subcore_barrier()
        pltpu.sync_copy(acc.at[pl.ds(s * RPT_N, RPT_N)],
                        out_hbm.at[c, pl.ds(s * RPT_N, RPT_N)])

    return cnt


def _pad_edges(src, dst):
    e = src.shape[0]
    rpt = -(-e // (NW * LW * 8)) * 8
    epad = NW * rpt * LW
    padi = jnp.arange(epad - e, dtype=jnp.int32)
    srcp = jnp.concatenate([src, padi % 16])
    dstp = jnp.concatenate([dst, N + (padi % ACC_PAD)])
    return srcp.reshape(NW * rpt, LW), dstp.reshape(NW * rpt, LW), rpt


@functools.cache
def _layer_tc(di, do):
    nci, nco = di // 128, do // 128
    nrb = N // RB

    def body(*refs):
        agg_refs = refs[:nci]
        x_refs = refs[nci:2 * nci]
        cnt_ref = refs[2 * nci]
        wl_ref, wr_ref, bl_ref = refs[2 * nci + 1:2 * nci + 4]
        out_refs = refs[2 * nci + 4:]
        cntv = cnt_ref[0, :, 0:1] + cnt_ref[1, :, 0:1]
        recip = 1.0 / jnp.maximum(cntv, 1.0)
        aggf = jnp.concatenate([r[0] + r[1] for r in agg_refs], axis=1) * recip
        xf = jnp.concatenate([r[...] for r in x_refs], axis=1)
        out = (lax.dot_general(aggf, wl_ref[...], (((1,), (1,)), ((), ())),
                               preferred_element_type=jnp.float32)
               + lax.dot_general(xf, wr_ref[...], (((1,), (1,)), ((), ())),
                                 preferred_element_type=jnp.float32)
               + bl_ref[...])
        nrm = jnp.sqrt(jnp.sum(out * out, axis=1, keepdims=True))
        out = jnp.maximum(out / jnp.maximum(nrm, 1e-12), 0.0)
        for co in range(nco):
            out_refs[co][...] = out[:, co * 128:(co + 1) * 128]

    in_specs = (
        [pl.BlockSpec((2, RB, 128), lambda r: (0, r, 0))] * nci
        + [pl.BlockSpec((RB, 128), lambda r: (r, 0))] * nci
        + [pl.BlockSpec((2, RB, 128), lambda r: (0, r, 0)),
           pl.BlockSpec((do, di), lambda r: (0, 0)),
           pl.BlockSpec((do, di), lambda r: (0, 0)),
           pl.BlockSpec((1, do), lambda r: (0, 0))]
    )
    out_specs = [pl.BlockSpec((RB, 128), lambda r: (r, 0))] * nco
    return pl.pallas_call(
        body,
        grid=(nrb,),
        in_specs=in_specs,
        out_specs=out_specs,
        out_shape=[jax.ShapeDtypeStruct((N, 128), jnp.float32)] * nco,
    )


@functools.cache
def _attn(nci):
    nrb = N // RB

    def body(q_ref, bi_ref, *refs):
        x_refs = refs[:nci]
        r_ref = refs[nci]
        eq_scr, m_scr = refs[nci + 1], refs[nci + 2]
        p = pl.program_id(0)
        r = pl.program_id(1)
        msk = bi_ref[...] == lax.broadcasted_iota(jnp.int32, (RB, B), 1)

        @pl.when(p == 0)
        def _():
            xb = jnp.concatenate([xr[...] for xr in x_refs], axis=1)
            eqb = lax.dot_general(xb, q_ref[...], (((1,), (1,)), ((), ())),
                                  preferred_element_type=jnp.float32)
            eq_scr[pl.ds(r * RB, RB), :] = eqb
            em = jnp.where(msk, eqb, -jnp.inf)
            pmax = jnp.max(em, axis=0, keepdims=True)

            @pl.when(r == 0)
            def _():
                m_scr[...] = jnp.broadcast_to(pmax, (8, B))

            @pl.when(r > 0)
            def _():
                m_scr[...] = jnp.maximum(m_scr[...], pmax)

        @pl.when(p == 1)
        def _():
            m = m_scr[0:1, :]
            mm = jnp.where(m < -3e38, 0.0, m)
            eqb = eq_scr[pl.ds(r * RB, RB), :]
            w = jnp.where(msk, jnp.exp(eqb - mm), 0.0)
            xb = jnp.concatenate([xr[...] for xr in x_refs]
                                 + [jnp.ones((RB, 128), jnp.float32)], axis=1)
            rpart = lax.dot_general(w, xb, (((0,), (0,)), ((), ())),
                                    preferred_element_type=jnp.float32)

            @pl.when(r == 0)
            def _():
                r_ref[...] = rpart

            @pl.when(r > 0)
            def _():
                r_ref[...] = r_ref[...] + rpart

            @pl.when(r == nrb - 1)
            def _():
                v = r_ref[...]
                den = jnp.maximum(v[:, nci * 128:nci * 128 + 1], 1e-16)
                r_ref[...] = v / den

    in_specs = (
        [pl.BlockSpec((B, D), lambda p, r: (0, 0)),
         pl.BlockSpec((RB, 1), lambda p, r: (r, 0))]
        + [pl.BlockSpec((RB, 128), lambda p, r: (r, 0))] * nci
    )
    out_specs = [pl.BlockSpec((B, (nci + 1) * 128), lambda p, r: (0, 0))]
    return pl.pallas_call(
        body,
        grid=(2, nrb),
        in_specs=in_specs,
        out_specs=out_specs,
        out_shape=[jax.ShapeDtypeStruct((B, (nci + 1) * 128), jnp.float32)],
        scratch_shapes=[pltpu.VMEM((N, B), jnp.float32),
                        pltpu.VMEM((8, B), jnp.float32)],
    )


def _sigmoid(x):
    return 1.0 / (1.0 + jnp.exp(-x))


def _lstm_body(q_ref, r_ref, c_ref, wih_ref, whh_ref, bih_ref, bhh_ref,
               h_out, c_out):
    q = q_ref[...]
    rr = r_ref[...]
    gates = []
    for g in range(4):
        wih = wih_ref[g]
        gate = (lax.dot_general(q, wih[:, :D], (((1,), (1,)), ((), ())),
                                preferred_element_type=jnp.float32)
                + lax.dot_general(rr, wih[:, D:], (((1,), (1,)), ((), ())),
                                  preferred_element_type=jnp.float32)
                + lax.dot_general(q, whh_ref[g], (((1,), (1,)), ((), ())),
                                  preferred_element_type=jnp.float32)
                + bih_ref[g:g + 1, :] + bhh_ref[g:g + 1, :])
        gates.append(gate)
    ig, fg, gg, og = gates
    c_new = _sigmoid(fg) * c_ref[...] + _sigmoid(ig) * jnp.tanh(gg)
    h_out[...] = _sigmoid(og) * jnp.tanh(c_new)
    c_out[...] = c_new


_lstm = pl.pallas_call(
    _lstm_body,
    out_shape=[jax.ShapeDtypeStruct((B, D), jnp.float32),
               jax.ShapeDtypeStruct((B, D), jnp.float32)],
)


def _heads_body(q_ref, r_ref, wmu_ref, bmu_ref, wlv_ref, blv_ref, eps_ref,
                z_ref, zmu_ref, zlv_ref):
    q = q_ref[...]
    rr = r_ref[...]

    def head(w_ref, b_ref):
        return (lax.dot_general(q, w_ref[:, :D], (((1,), (1,)), ((), ())),
                                preferred_element_type=jnp.float32)
                + lax.dot_general(rr, w_ref[:, D:], (((1,), (1,)), ((), ())),
                                  preferred_element_type=jnp.float32)
                + b_ref[...])

    zmu = head(wmu_ref, bmu_ref)
    zlv = head(wlv_ref, blv_ref)
    zmu_ref[...] = zmu
    zlv_ref[...] = zlv
    z_ref[...] = eps_ref[...] * jnp.exp(0.5 * zlv) + zmu


_heads = pl.pallas_call(
    _heads_body,
    out_shape=[jax.ShapeDtypeStruct((B, EMB), jnp.float32)] * 3,
)


def kernel(x, edge_attr, edge_index, batch_index, params):
    f32 = jnp.float32
    src2d, dst2d, rpt = _pad_edges(edge_index[0], edge_index[1])

    cnt2 = _sc_cnt(rpt)(dst2d)
    slabs = [x]
    for i, (di, do) in enumerate(DIMS):
        aggs = _sc_agg(rpt, len(slabs))(*slabs, src2d, dst2d)
        if not isinstance(aggs, (tuple, list)):
            aggs = [aggs]
        slabs = _layer_tc(di, do)(
            *aggs, *slabs, cnt2,
            params[f"Wl{i}"], params[f"Wr{i}"],
            params[f"bl{i}"].reshape(1, do))

    bi2d = batch_index.reshape(N, 1)
    wih4 = params["W_ih"].reshape(4, D, 2 * D)
    whh4 = params["W_hh"].reshape(4, D, D)
    bih4 = params["b_ih"].reshape(4, D)
    bhh4 = params["b_hh"].reshape(4, D)
    nci = len(slabs)

    q = jnp.zeros((B, D), f32)
    r = jnp.zeros((B, D), f32)
    cst = jnp.zeros((B, D), f32)
    for _ in range(4):
        q, cst = _lstm(q, r, cst, wih4, whh4, bih4, bhh4)
        (rfull,) = _attn(nci)(q, bi2d, *slabs)
        r = rfull[:, :D]

    eps = jax.random.normal(jax.random.key(42), (B, EMB), f32)
    z, zmu, zlv = _heads(q, r, params["Wmu"], params["bmu"].reshape(1, EMB),
                         params["Wlv"], params["blv"].reshape(1, EMB), eps)
    return (z, zmu, zlv)


# speedup vs baseline: 1.2030x; 1.0002x over previous
import functools

import jax
import jax.numpy as jnp
from jax import lax
from jax.experimental import pallas as pl
from jax.experimental.pallas import tpu as pltpu
from jax.experimental.pallas import tpu_sc as plsc

N = 10000
B = 64
DIMS = [(128, 128), (128, 256), (256, 512), (512, 1024)]
D = 1024
EMB = 128

NC = 2
NS = 16
NW = NC * NS
LW = 128
ACC_PAD = 16
ACC_ROWS = 10240
ZR = 320
RPT_N = ACC_ROWS // NS
RB = 1000


@functools.cache
def _sc_agg(rpt, nchunks):
    mesh = plsc.VectorSubcoreMesh(core_axis_name="c", subcore_axis_name="s")

    @functools.partial(
        pl.kernel,
        out_type=[jax.ShapeDtypeStruct((NC, ACC_ROWS, 128), jnp.float32)]
        * nchunks,
        mesh=mesh,
        scratch_types=[
            pltpu.VMEM((rpt // 2, LW), jnp.int32),
            pltpu.VMEM((rpt // 2, LW), jnp.int32),
            pltpu.VMEM((LW, 128), jnp.float32),
            pltpu.VMEM((LW, 128), jnp.float32),
            pltpu.VMEM_SHARED((ACC_ROWS, 128), jnp.float32),
            pltpu.SemaphoreType.DMA,
            pltpu.SemaphoreType.DMA,
        ],
    )
    def agg(*refs):
        x_hbms = refs[:nchunks]
        src_hbm, dst_hbm = refs[nchunks], refs[nchunks + 1]
        out_hbms = refs[nchunks + 2:2 * nchunks + 2]
        (src_v, dst_v, rows_a, rows_b, acc,
         sem_a, sem_b) = refs[2 * nchunks + 2:]
        c = lax.axis_index("c")
        s = lax.axis_index("s")
        wid = s * NC + c
        half = rpt // 2
        npairs = half // 2

        zero16 = jnp.zeros((16,), jnp.float32)

        def zrow(i, carry):
            for l in range(128 // 16):
                rows_a[i, pl.ds(l * 16, 16)] = zero16
            return carry

        def wait_g(x_hbm, buf, sem):
            pltpu.make_async_copy(x_hbm.at[pl.ds(0, LW)], buf, sem).wait()

        for ci, (x_hbm, out_hbm) in enumerate(zip(x_hbms, out_hbms)):
            if ci == 0:
                lax.fori_loop(0, LW, zrow, 0)
            for k in range(RPT_N // LW):
                pltpu.sync_copy(rows_a, acc.at[pl.ds(s * RPT_N + k * LW, LW)])
            plsc.subcore_barrier()

            for h in range(2):
                base = wid * rpt + h * half
                pltpu.sync_copy(src_hbm.at[pl.ds(base, half)], src_v)
                pltpu.sync_copy(dst_hbm.at[pl.ds(base, half)], dst_v)
                pltpu.async_copy(x_hbm.at[src_v.at[0]], rows_a, sem_a)
                pltpu.async_copy(x_hbm.at[src_v.at[1]], rows_b, sem_b)

                def body(t, carry):
                    ja = t * 2
                    wait_g(x_hbm, rows_a, sem_a)
                    pltpu.sync_copy(rows_a, acc.at[dst_v.at[ja]], add=True)

                    @pl.when(t + 1 < npairs)
                    def _():
                        pltpu.async_copy(x_hbm.at[src_v.at[ja + 2]], rows_a,
                                         sem_a)

                    wait_g(x_hbm, rows_b, sem_b)
                    pltpu.sync_copy(rows_b, acc.at[dst_v.at[ja + 1]], add=True)

                    @pl.when(t + 1 < npairs)
                    def _():
                        pltpu.async_copy(x_hbm.at[src_v.at[ja + 3]], rows_b,
                                         sem_b)

                    return carry

                lax.fori_loop(0, npairs, body, 0)
            plsc.subcore_barrier()
            pltpu.sync_copy(acc.at[pl.ds(s * RPT_N, RPT_N)],
                            out_hbm.at[c, pl.ds(s * RPT_N, RPT_N)])
            if ci + 1 < nchunks:
                lax.fori_loop(0, LW, zrow, 0)
                plsc.subcore_barrier()

    return agg


@functools.cache
def _sc_cnt(rpt):
    mesh = plsc.VectorSubcoreMesh(core_axis_name="c", subcore_axis_name="s")

    @functools.partial(
        pl.kernel,
        out_type=jax.ShapeDtypeStruct((NC, ACC_ROWS, 128), jnp.float32),
        mesh=mesh,
        scratch_types=[
            pltpu.VMEM((rpt, LW), jnp.int32),
            pltpu.VMEM((LW, 128), jnp.float32),
            pltpu.VMEM_SHARED((ACC_ROWS, 128), jnp.float32),
        ],
    )
    def cnt(dst_hbm, out_hbm, dst_v, ones_v, acc):
        c = lax.axis_index("c")
        s = lax.axis_index("s")
        wid = s * NC + c

        def fill(val):
            v = jnp.full((16,), val, jnp.float32)

            def frow(i, carry):
                for l in range(128 // 16):
                    ones_v[i, pl.ds(l * 16, 16)] = v
                return carry

            lax.fori_loop(0, LW, frow, 0)

        fill(0.0)
        for k in range(RPT_N // LW):
            pltpu.sync_copy(ones_v, acc.at[pl.ds(s * RPT_N + k * LW, LW)])
        fill(1.0)
        pltpu.sync_copy(dst_hbm.at[pl.ds(wid * rpt, rpt)], dst_v)
        plsc.subcore_barrier()

        def body(j, carry):
            pltpu.sync_copy(ones_v, acc.at[dst_v.at[j]], add=True)
            return carry

        lax.fori_loop(0, rpt, body, 0)
        plsc.subcore_barrier()
        pltpu.sync_copy(acc.at[pl.ds(s * RPT_N, RPT_N)],
                        out_hbm.at[c, pl.ds(s * RPT_N, RPT_N)])

    return cnt


def _pad_edges(src, dst):
    e = src.shape[0]
    rpt = -(-e // (NW * LW * 8)) * 8
    epad = NW * rpt * LW
    padi = jnp.arange(epad - e, dtype=jnp.int32)
    srcp = jnp.concatenate([src, padi % 16])
    dstp = jnp.concatenate([dst, N + (padi % ACC_PAD)])
    return srcp.reshape(NW * rpt, LW), dstp.reshape(NW * rpt, LW), rpt


@functools.cache
def _layer_tc(di, do):
    nci, nco = di // 128, do // 128
    nrb = N // RB

    def body(*refs):
        agg_refs = refs[:nci]
        x_refs = refs[nci:2 * nci]
        cnt_ref = refs[2 * nci]
        wl_ref, wr_ref, bl_ref = refs[2 * nci + 1:2 * nci + 4]
        out_refs = refs[2 * nci + 4:]
        cntv = cnt_ref[0, :, 0:1] + cnt_ref[1, :, 0:1]
        recip = 1.0 / jnp.maximum(cntv, 1.0)
        aggf = jnp.concatenate([r[0] + r[1] for r in agg_refs], axis=1) * recip
        xf = jnp.concatenate([r[...] for r in x_refs], axis=1)
        out = (lax.dot_general(aggf, wl_ref[...], (((1,), (1,)), ((), ())),
                               preferred_element_type=jnp.float32)
               + lax.dot_general(xf, wr_ref[...], (((1,), (1,)), ((), ())),
                                 preferred_element_type=jnp.float32)
               + bl_ref[...])
        nrm = jnp.sqrt(jnp.sum(out * out, axis=1, keepdims=True))
        out = jnp.maximum(out / jnp.maximum(nrm, 1e-12), 0.0)
        for co in range(nco):
            out_refs[co][...] = out[:, co * 128:(co + 1) * 128]

    in_specs = (
        [pl.BlockSpec((2, RB, 128), lambda r: (0, r, 0))] * nci
        + [pl.BlockSpec((RB, 128), lambda r: (r, 0))] * nci
        + [pl.BlockSpec((2, RB, 128), lambda r: (0, r, 0)),
           pl.BlockSpec((do, di), lambda r: (0, 0)),
           pl.BlockSpec((do, di), lambda r: (0, 0)),
           pl.BlockSpec((1, do), lambda r: (0, 0))]
    )
    out_specs = [pl.BlockSpec((RB, 128), lambda r: (r, 0))] * nco
    return pl.pallas_call(
        body,
        grid=(nrb,),
        in_specs=in_specs,
        out_specs=out_specs,
        out_shape=[jax.ShapeDtypeStruct((N, 128), jnp.float32)] * nco,
    )


@functools.cache
def _attn(nci):
    nrb = N // RB

    def body(q_ref, bi_ref, *refs):
        x_refs = refs[:nci]
        r_ref = refs[nci]
        eq_scr, m_scr = refs[nci + 1], refs[nci + 2]
        p = pl.program_id(0)
        r = pl.program_id(1)
        msk = bi_ref[...] == lax.broadcasted_iota(jnp.int32, (RB, B), 1)

        @pl.when(p == 0)
        def _():
            xb = jnp.concatenate([xr[...] for xr in x_refs], axis=1)
            eqb = lax.dot_general(xb, q_ref[...], (((1,), (1,)), ((), ())),
                                  preferred_element_type=jnp.float32)
            eq_scr[pl.ds(r * RB, RB), :] = eqb
            em = jnp.where(msk, eqb, -jnp.inf)
            pmax = jnp.max(em, axis=0, keepdims=True)

            @pl.when(r == 0)
            def _():
                m_scr[...] = jnp.broadcast_to(pmax, (8, B))

            @pl.when(r > 0)
            def _():
                m_scr[...] = jnp.maximum(m_scr[...], pmax)

        @pl.when(p == 1)
        def _():
            m = m_scr[0:1, :]
            mm = jnp.where(m < -3e38, 0.0, m)
            eqb = eq_scr[pl.ds(r * RB, RB), :]
            w = jnp.where(msk, jnp.exp(eqb - mm), 0.0)
            xb = jnp.concatenate([xr[...] for xr in x_refs]
                                 + [jnp.ones((RB, 128), jnp.float32)], axis=1)
            rpart = lax.dot_general(w, xb, (((0,), (0,)), ((), ())),
                                    preferred_element_type=jnp.float32)

            @pl.when(r == 0)
            def _():
                r_ref[...] = rpart

            @pl.when(r > 0)
            def _():
                r_ref[...] = r_ref[...] + rpart

            @pl.when(r == nrb - 1)
            def _():
                v = r_ref[...]
                den = jnp.maximum(v[:, nci * 128:nci * 128 + 1], 1e-16)
                r_ref[...] = v / den

    in_specs = (
        [pl.BlockSpec((B, D), lambda p, r: (0, 0)),
         pl.BlockSpec((RB, 1), lambda p, r: (r, 0))]
        + [pl.BlockSpec((RB, 128), lambda p, r: (r, 0))] * nci
    )
    out_specs = [pl.BlockSpec((B, (nci + 1) * 128), lambda p, r: (0, 0))]
    return pl.pallas_call(
        body,
        grid=(2, nrb),
        in_specs=in_specs,
        out_specs=out_specs,
        out_shape=[jax.ShapeDtypeStruct((B, (nci + 1) * 128), jnp.float32)],
        scratch_shapes=[pltpu.VMEM((N, B), jnp.float32),
                        pltpu.VMEM((8, B), jnp.float32)],
    )


def _sigmoid(x):
    return 1.0 / (1.0 + jnp.exp(-x))


def _lstm_body(q_ref, r_ref, c_ref, wih_ref, whh_ref, bih_ref, bhh_ref,
               h_out, c_out):
    q = q_ref[...]
    rr = r_ref[...]
    gates = []
    for g in range(4):
        wih = wih_ref[g]
        gate = (lax.dot_general(q, wih[:, :D], (((1,), (1,)), ((), ())),
                                preferred_element_type=jnp.float32)
                + lax.dot_general(rr, wih[:, D:], (((1,), (1,)), ((), ())),
                                  preferred_element_type=jnp.float32)
                + lax.dot_general(q, whh_ref[g], (((1,), (1,)), ((), ())),
                                  preferred_element_type=jnp.float32)
                + bih_ref[g:g + 1, :] + bhh_ref[g:g + 1, :])
        gates.append(gate)
    ig, fg, gg, og = gates
    c_new = _sigmoid(fg) * c_ref[...] + _sigmoid(ig) * jnp.tanh(gg)
    h_out[...] = _sigmoid(og) * jnp.tanh(c_new)
    c_out[...] = c_new


_lstm = pl.pallas_call(
    _lstm_body,
    out_shape=[jax.ShapeDtypeStruct((B, D), jnp.float32),
               jax.ShapeDtypeStruct((B, D), jnp.float32)],
)


def _heads_body(q_ref, r_ref, wmu_ref, bmu_ref, wlv_ref, blv_ref, eps_ref,
                z_ref, zmu_ref, zlv_ref):
    q = q_ref[...]
    rr = r_ref[...]

    def head(w_ref, b_ref):
        return (lax.dot_general(q, w_ref[:, :D], (((1,), (1,)), ((), ())),
                                preferred_element_type=jnp.float32)
                + lax.dot_general(rr, w_ref[:, D:], (((1,), (1,)), ((), ())),
                                  preferred_element_type=jnp.float32)
                + b_ref[...])

    zmu = head(wmu_ref, bmu_ref)
    zlv = head(wlv_ref, blv_ref)
    zmu_ref[...] = zmu
    zlv_ref[...] = zlv
    z_ref[...] = eps_ref[...] * jnp.exp(0.5 * zlv) + zmu


_heads = pl.pallas_call(
    _heads_body,
    out_shape=[jax.ShapeDtypeStruct((B, EMB), jnp.float32)] * 3,
)


def kernel(x, edge_attr, edge_index, batch_index, params):
    f32 = jnp.float32
    src2d, dst2d, rpt = _pad_edges(edge_index[0], edge_index[1])

    cnt2 = _sc_cnt(rpt)(dst2d)
    slabs = [x]
    for i, (di, do) in enumerate(DIMS):
        aggs = _sc_agg(rpt, len(slabs))(*slabs, src2d, dst2d)
        if not isinstance(aggs, (tuple, list)):
            aggs = [aggs]
        slabs = _layer_tc(di, do)(
            *aggs, *slabs, cnt2,
            params[f"Wl{i}"], params[f"Wr{i}"],
            params[f"bl{i}"].reshape(1, do))

    bi2d = batch_index.reshape(N, 1)
    wih4 = params["W_ih"].reshape(4, D, 2 * D)
    whh4 = params["W_hh"].reshape(4, D, D)
    bih4 = params["b_ih"].reshape(4, D)
    bhh4 = params["b_hh"].reshape(4, D)
    nci = len(slabs)

    q = jnp.zeros((B, D), f32)
    r = jnp.zeros((B, D), f32)
    cst = jnp.zeros((B, D), f32)
    for _ in range(4):
        q, cst = _lstm(q, r, cst, wih4, whh4, bih4, bhh4)
        (rfull,) = _attn(nci)(q, bi2d, *slabs)
        r = rfull[:, :D]

    eps = jax.random.normal(jax.random.key(42), (B, EMB), f32)
    z, zmu, zlv = _heads(q, r, params["Wmu"], params["bmu"].reshape(1, EMB),
                         params["Wlv"], params["blv"].reshape(1, EMB), eps)
    return (z, zmu, zlv)
